# baseline (speedup 1.0000x reference)
import functools

import jax
import jax.numpy as jnp
from jax import lax
from jax.experimental import pallas as pl
from jax.experimental.pallas import tpu as pltpu
from jax.experimental.pallas import tpu_sc as plsc

_T = 1024
_S = 32
_F = 512
_COLS = 1 + _S * _T
_L = 16
_NC = 2
_NS = 16
_NW = _NC * _NS
_RPW = _T // _NW
_RPC = _T // _NS
_HROWS = 32


def _sc_body(x_hbm, wb_hbm, out_hbm,
             xbuf, wbuf, myscores, scores_sh, scores_v,
             buf_main, buf_first, sem):
    cid = lax.axis_index("c")
    sid = lax.axis_index("s")
    lane = lax.iota(jnp.int32, _L)

    pltpu.sync_copy(wb_hbm, wbuf.at[pl.ds(0, _F + 1)])
    bval = wbuf[pl.ds(_F, _L)][0]
    r0 = sid * _RPC
    for h in range(_RPC // _HROWS):
        pltpu.sync_copy(x_hbm.at[pl.ds(r0 + h * _HROWS, _HROWS), :], xbuf)
        for g in range(_HROWS // _L):
            def row_dot(rr, svec):
                acc = jnp.zeros((_L,), jnp.float32)
                for k in range(_F // _L):
                    acc = acc + (xbuf[g * _L + rr, pl.ds(k * _L, _L)]
                                 * wbuf[pl.ds(k * _L, _L)])
                s = jnp.sum(acc) + bval
                return jnp.where(lane == rr, s, svec)

            svec = lax.fori_loop(0, _L, row_dot, jnp.zeros((_L,), jnp.float32))
            myscores[pl.ds(h * _HROWS + g * _L, _L)] = svec

    pltpu.sync_copy(myscores, scores_sh.at[pl.ds(r0, _RPC)])
    plsc.subcore_barrier()
    pltpu.sync_copy(scores_sh, scores_v.at[pl.ds(0, _T)])

    def amax_body(c, carry):
        runmax, runidx = carry
        v = scores_v[pl.ds(c * _L, _L)]
        idxv = c * _L + lane
        upd = v > runmax
        return jnp.where(upd, v, runmax), jnp.where(upd, idxv, runidx)

    runmax, runidx = lax.fori_loop(
        0, _T // _L, amax_body,
        (jnp.full((_L,), -jnp.inf, jnp.float32), jnp.zeros((_L,), jnp.int32)))
    gm = jnp.max(runmax)
    idx = jnp.min(jnp.where(runmax == gm, runidx, jnp.int32(_T)))
    sel = lax.rem(idx + jnp.int32(1), jnp.int32(_T))

    wid = sid * _NC + cid
    row0 = wid * _RPW
    ninf = jnp.float32(-jnp.inf)
    one = jnp.float32(1.0)

    def build_row(r, carry):
        s = scores_v[pl.ds(row0 + r, _L)][0]
        for j in range(_T // _L):
            colv = j * _L + lane
            hit = colv == sel
            val = s * jnp.where(hit, ninf, one)
            buf_main[r, pl.ds(j * _L, _L)] = val
            if j == 0:
                val0 = s * jnp.where(hit & (colv > 0), ninf, one)
                buf_first[r, pl.ds(0, _L)] = val0
            else:
                buf_first[r, pl.ds(j * _L, _L)] = val
        return carry

    lax.fori_loop(0, _RPW, build_row, 0)

    rows = pl.ds(row0, _RPW)
    pltpu.sync_copy(buf_first, out_hbm.at[rows, pl.ds(0, _T)])

    def dma_chunk(k, carry):
        pltpu.async_copy(buf_main, out_hbm.at[rows, pl.ds(k * _T, _T)], sem).wait()
        return carry

    lax.fori_loop(1, _S, dma_chunk, 0)


def _tc_tail_fix(in_ref, out_ref):
    out_ref[...] = in_ref[...]


def _round_bf16(v):
    u = jax.lax.bitcast_convert_type(v, jnp.uint32)
    r = (u + jnp.uint32(0x7FFF) + ((u >> 16) & jnp.uint32(1))) & jnp.uint32(0xFFFF0000)
    return jax.lax.bitcast_convert_type(r, jnp.float32)


def kernel(x, W, b):
    xr = _round_bf16(x)
    wb = jnp.concatenate([_round_bf16(W).reshape(-1), b])
    mesh = plsc.VectorSubcoreMesh(core_axis_name="c", subcore_axis_name="s")
    k = functools.partial(
        pl.kernel,
        mesh=mesh,
        compiler_params=pltpu.CompilerParams(use_tc_tiling_on_sc=False,
                                             needs_layout_passes=False),
        out_type=jax.ShapeDtypeStruct((_T, _COLS), jnp.float32),
        scratch_types=[
            pltpu.VMEM((_HROWS, _F), jnp.float32),
            pltpu.VMEM((_F + _L,), jnp.float32),
            pltpu.VMEM((_RPC,), jnp.float32),
            pltpu.VMEM_SHARED((_T,), jnp.float32),
            pltpu.VMEM((_T + _L,), jnp.float32),
            pltpu.VMEM((_RPW, _T), jnp.float32),
            pltpu.VMEM((_RPW, _T), jnp.float32),
            pltpu.SemaphoreType.DMA,
        ],
    )(_sc_body)
    out = k(xr, wb)
    return pl.pallas_call(
        _tc_tail_fix,
        grid=(1,),
        in_specs=[pl.BlockSpec((_T, 128), lambda i: (0, _T // 128))],
        out_specs=pl.BlockSpec((_T, 128), lambda i: (0, (_S * _T) // 128)),
        out_shape=jax.ShapeDtypeStruct((_T, _COLS), jnp.float32),
        input_output_aliases={0: 0},
    )(out)


# speedup vs baseline: 5.4622x; 5.4622x over previous
import functools

import jax
import jax.numpy as jnp
from jax import lax
from jax.experimental import pallas as pl
from jax.experimental.pallas import tpu as pltpu
from jax.experimental.pallas import tpu_sc as plsc

_T = 1024
_S = 32
_F = 512
_COLS = 1 + _S * _T
_L = 16
_NS = 16
_RPC = _T // _NS
_HROWS = 32


def _sc_body(x_hbm, wb_hbm, out_hbm, xbuf, wbuf, myscores, scores_sh, scores_v):
    cid = lax.axis_index("c")
    sid = lax.axis_index("s")
    lane = lax.iota(jnp.int32, _L)

    pltpu.sync_copy(wb_hbm, wbuf.at[pl.ds(0, _F + 1)])
    bval = wbuf[pl.ds(_F, _L)][0]
    r0 = sid * _RPC
    for h in range(_RPC // _HROWS):
        pltpu.sync_copy(x_hbm.at[pl.ds(r0 + h * _HROWS, _HROWS), :], xbuf)
        for g in range(_HROWS // _L):
            def row_dot(rr, svec):
                acc = jnp.zeros((_L,), jnp.float32)
                for k in range(_F // _L):
                    acc = acc + (xbuf[g * _L + rr, pl.ds(k * _L, _L)]
                                 * wbuf[pl.ds(k * _L, _L)])
                s = jnp.sum(acc) + bval
                return jnp.where(lane == rr, s, svec)

            svec = lax.fori_loop(0, _L, row_dot, jnp.zeros((_L,), jnp.float32))
            myscores[pl.ds(h * _HROWS + g * _L, _L)] = svec

    pltpu.sync_copy(myscores, scores_sh.at[pl.ds(r0, _RPC)])
    plsc.subcore_barrier()
    pltpu.sync_copy(scores_sh, scores_v.at[pl.ds(0, _T)])

    def amax_body(c, carry):
        runmax, runidx = carry
        v = scores_v[pl.ds(c * _L, _L)]
        idxv = c * _L + lane
        upd = v > runmax
        return jnp.where(upd, v, runmax), jnp.where(upd, idxv, runidx)

    runmax, runidx = lax.fori_loop(
        0, _T // _L, amax_body,
        (jnp.full((_L,), -jnp.inf, jnp.float32), jnp.zeros((_L,), jnp.int32)))
    gm = jnp.max(runmax)
    idx = jnp.min(jnp.where(runmax == gm, runidx, jnp.int32(_T)))
    sel = lax.rem(idx + jnp.int32(1), jnp.int32(_T))
    scores_v[pl.ds(_T, _L)] = (lane * 0 + sel).astype(jnp.float32)

    @pl.when((cid == 0) & (sid == 0))
    def _publish():
        pltpu.sync_copy(scores_v, out_hbm)


def _tc_body(scores_ref, sel_ref, out_ref):
    j = pl.program_id(0)
    sel = sel_ref[0, 0]
    cols = jax.lax.broadcasted_iota(jnp.int32, (1, _T), 1)
    m = jnp.where(cols == sel, jnp.float32(-jnp.inf), jnp.float32(1.0))
    m = jnp.where((j == 0) & (cols == 0), jnp.float32(1.0), m)
    out_ref[...] = scores_ref[...] * m


def _round_bf16(v):
    u = jax.lax.bitcast_convert_type(v, jnp.uint32)
    r = (u + jnp.uint32(0x7FFF) + ((u >> 16) & jnp.uint32(1))) & jnp.uint32(0xFFFF0000)
    return jax.lax.bitcast_convert_type(r, jnp.float32)


def kernel(x, W, b):
    xr = _round_bf16(x)
    wb = jnp.concatenate([_round_bf16(W).reshape(-1), b])
    mesh = plsc.VectorSubcoreMesh(core_axis_name="c", subcore_axis_name="s")
    sc = functools.partial(
        pl.kernel,
        mesh=mesh,
        compiler_params=pltpu.CompilerParams(use_tc_tiling_on_sc=False,
                                             needs_layout_passes=False),
        out_type=jax.ShapeDtypeStruct((_T + _L,), jnp.float32),
        scratch_types=[
            pltpu.VMEM((_HROWS, _F), jnp.float32),
            pltpu.VMEM((_F + _L,), jnp.float32),
            pltpu.VMEM((_RPC,), jnp.float32),
            pltpu.VMEM_SHARED((_T,), jnp.float32),
            pltpu.VMEM((_T + _L,), jnp.float32),
        ],
    )(_sc_body)
    head = sc(xr, wb)
    scores = head[:_T].reshape(_T, 1)
    sel = head[_T:_T + 1].astype(jnp.int32).reshape(1, 1)

    grid = (_COLS + _T - 1) // _T
    return pl.pallas_call(
        _tc_body,
        grid=(grid,),
        in_specs=[
            pl.BlockSpec((_T, 1), lambda j: (0, 0)),
            pl.BlockSpec(memory_space=pltpu.SMEM),
        ],
        out_specs=pl.BlockSpec((_T, _T), lambda j: (0, j)),
        out_shape=jax.ShapeDtypeStruct((_T, _COLS), jnp.float32),
    )(scores, sel)
